# Initial kernel scaffold; baseline (speedup 1.0000x reference)
import jax
import jax.numpy as jnp
from jax.experimental import pallas as pl


def kernel(input, target):
    raise NotImplementedError("write your pallas kernel here")


# speedup vs baseline: 28.3681x; 28.3681x over previous
import functools

import jax
import jax.numpy as jnp
from jax import lax
from jax.experimental import pallas as pl
from jax.experimental.pallas import tpu as pltpu
from jax.experimental.pallas import tpu_sc as plsc

B = 8
N = 512 * 512
NTOT = B * N
NW = 32
E = NTOT // NW
CHUNK = 16384
NCHUNK = E // CHUNK

_P = (3.5094790291623212e-06, 0.9997923970222473, -0.49697765707969666,
      0.3145897686481476, -0.18878164887428284, 0.08172616362571716,
      -0.0172079149633646)


def _log1p_poly(e):
    acc = jnp.full_like(e, _P[6])
    for c in (_P[5], _P[4], _P[3], _P[2], _P[1], _P[0]):
        acc = acc * e + c
    return acc


def _softplus(xv):
    e = jnp.exp(-jnp.abs(xv))
    return jnp.maximum(xv, jnp.float32(0.0)) + _log1p_poly(e)


_MESH = plsc.VectorSubcoreMesh(core_axis_name="c", subcore_axis_name="s",
                               num_cores=2, num_subcores=16)


def _sc_sums_body(x_hbm, t_hbm, out_hbm, xb, tb, ob):
    c = lax.axis_index("c")
    s = lax.axis_index("s")
    wid = s * 2 + c
    base = wid * E

    def chunk_loop(ci, carry):
        acc_l, acc_t = carry
        off = base + ci * CHUNK
        pltpu.sync_copy(x_hbm.at[pl.ds(off, CHUNK)], xb)
        pltpu.sync_copy(t_hbm.at[pl.ds(off, CHUNK)], tb)

        def vloop(i, carry2):
            al, at = carry2
            xv = xb[pl.ds(i * 16, 16)]
            tv = tb[pl.ds(i * 16, 16)]
            loss = _softplus(xv) - xv * tv
            return al + loss, at + tv

        return lax.fori_loop(0, CHUNK // 16, vloop, (acc_l, acc_t),
                             unroll=4)

    z = jnp.zeros((16,), jnp.float32)
    acc_l, acc_t = lax.fori_loop(0, NCHUNK, chunk_loop, (z, z))
    ob[0, :] = acc_l
    ob[1, :] = acc_t
    pltpu.sync_copy(ob, out_hbm.at[wid])


RCHUNK = 16384
RNCHUNK = N // RCHUNK


def _key_u32(xv, tv):
    b = lax.bitcast_convert_type(xv, jnp.uint32)
    neg = (b >> jnp.uint32(31)) == jnp.uint32(1)
    key = jnp.where(neg, ~b, b | jnp.uint32(0x80000000))
    key = jnp.maximum(key, jnp.uint32(1))
    return jnp.where(tv == jnp.float32(1.0), jnp.uint32(0), key)


def _lane_sum(v):
    idx = lax.iota(jnp.int32, 16)
    for sh in (1, 2, 4, 8):
        rot = v.at[(idx + sh) & 15].get(mode="promise_in_bounds")
        v = v + rot
    return v


def _sc_topk_body(x_hbm, t_hbm, out_hbm, xb, tb, ob):
    vsum_f = _lane_sum
    vsum_i = _lane_sum

    c = lax.axis_index("c")
    s = lax.axis_index("s")
    wid = s * 2 + c
    samp = jnp.minimum(wid, B - 1)
    base = samp * N
    zi = jnp.zeros((16,), jnp.int32)
    zf = jnp.zeros((16,), jnp.float32)

    def count_pass(cand_incl):
        def chunk_loop(ci, acc):
            off = base + ci * RCHUNK
            pltpu.sync_copy(x_hbm.at[pl.ds(off, RCHUNK)], xb)
            pltpu.sync_copy(t_hbm.at[pl.ds(off, RCHUNK)], tb)

            def vloop(i, a):
                key = _key_u32(xb[pl.ds(i * 16, 16)], tb[pl.ds(i * 16, 16)])
                return a + jnp.where(key >= cand_incl, jnp.int32(1),
                                     jnp.int32(0))

            return lax.fori_loop(0, RCHUNK // 16, vloop, acc, unroll=4)

        acc = lax.fori_loop(0, RNCHUNK, chunk_loop, zi)
        return vsum_i(acc)

    def nw_chunk(ci, acc):
        off = base + ci * RCHUNK
        pltpu.sync_copy(t_hbm.at[pl.ds(off, RCHUNK)], tb)

        def vloop(i, a):
            return a + tb[pl.ds(i * 16, 16)]

        return lax.fori_loop(0, RCHUNK // 16, vloop, acc, unroll=4)

    nwv = lax.fori_loop(0, RNCHUNK, nw_chunk, zf)
    n_white_f = vsum_f(nwv)
    n_white = n_white_f.astype(jnp.int32)
    n_black = jnp.full((16,), N, jnp.int32) - n_white
    k = jnp.minimum(3 * n_white, n_black)

    def bit_step(j, prefix):
        bit = jnp.full((16,), 1, jnp.uint32) << (
            jnp.uint32(31) - j.astype(jnp.uint32))
        cand = prefix | bit
        cnt = count_pass(cand)
        return jnp.where(cnt >= k, cand, prefix)

    T = lax.fori_loop(0, 32, bit_step, jnp.zeros((16,), jnp.uint32))

    def fin_chunk(ci, carry):
        a_w, a_bs, a_bc = carry
        off = base + ci * RCHUNK
        pltpu.sync_copy(x_hbm.at[pl.ds(off, RCHUNK)], xb)
        pltpu.sync_copy(t_hbm.at[pl.ds(off, RCHUNK)], tb)

        def vloop(i, cc):
            aw, abs_, abc = cc
            xv = xb[pl.ds(i * 16, 16)]
            tv = tb[pl.ds(i * 16, 16)]
            sp = _softplus(xv)
            key = _key_u32(xv, tv)
            white = tv == jnp.float32(1.0)
            gt = key > T
            aw = aw + jnp.where(white, sp - xv, jnp.float32(0.0))
            abs_ = abs_ + jnp.where(gt, sp, jnp.float32(0.0))
            abc = abc + jnp.where(gt, jnp.float32(1.0), jnp.float32(0.0))
            return aw, abs_, abc

        return lax.fori_loop(0, RCHUNK // 16, vloop, (a_w, a_bs, a_bc),
                             unroll=4)

    aw, abs_, abc = lax.fori_loop(0, RNCHUNK, fin_chunk, (zf, zf, zf))
    sum_white = vsum_f(aw)
    sum_gt = vsum_f(abs_)
    cnt_gt = vsum_f(abc)

    tbits = jnp.where(T >= jnp.uint32(0x80000000), T & jnp.uint32(0x7FFFFFFF),
                      ~T)
    sp_tie = _softplus(lax.bitcast_convert_type(tbits, jnp.float32))
    n_tie = k.astype(jnp.float32) - cnt_gt
    sum_black = sum_gt + jnp.where(k > 0, n_tie * sp_tie,
                                   jnp.float32(0.0))

    ob[0, :] = sum_white + sum_black
    ob[1, :] = n_white_f + k.astype(jnp.float32)
    ob[2, :] = zf
    ob[3, :] = zf
    pltpu.sync_copy(ob, out_hbm.at[wid])


def _build_kernels(interpret=False):
    sums = pl.kernel(
        _sc_sums_body,
        out_type=jax.ShapeDtypeStruct((NW, 2, 16), jnp.float32),
        mesh=_MESH,
        scratch_types=[
            pltpu.VMEM((CHUNK,), jnp.float32),
            pltpu.VMEM((CHUNK,), jnp.float32),
            pltpu.VMEM((2, 16), jnp.float32),
        ],
        interpret=interpret,
    )
    topk = pl.kernel(
        _sc_topk_body,
        out_type=jax.ShapeDtypeStruct((NW, 4, 16), jnp.float32),
        mesh=_MESH,
        scratch_types=[
            pltpu.VMEM((RCHUNK,), jnp.float32),
            pltpu.VMEM((RCHUNK,), jnp.float32),
            pltpu.VMEM((4, 16), jnp.float32),
        ],
        interpret=interpret,
    )
    return sums, topk


_sc_sums, _sc_topk = _build_kernels()


def kernel(input, target):
    x = input.reshape(NTOT)
    t = target.reshape(NTOT)
    parts = _sc_sums(x, t)
    per_worker = jnp.sum(parts, axis=2)
    pw = per_worker.reshape(B, NW // B, 2)
    loss_sum = jnp.sum(pw[:, :, 0])
    n_white = jnp.sum(pw[:, :, 1], axis=1)
    n_black = jnp.float32(N) - n_white
    common = loss_sum / jnp.float32(NTOT)

    def rare():
        out = _sc_topk(x, t)
        sums = out[:B, 0, 0]
        cnts = out[:B, 1, 0]
        return jnp.sum(sums) / jnp.sum(cnts)

    pred = jnp.all(3.0 * n_white >= n_black)
    return lax.cond(pred, lambda: common, rare)


# speedup vs baseline: 44.3178x; 1.5622x over previous
import functools

import jax
import jax.numpy as jnp
from jax import lax
from jax.experimental import pallas as pl
from jax.experimental.pallas import tpu as pltpu
from jax.experimental.pallas import tpu_sc as plsc

B = 8
H = 512
W = 512
N = H * W
NTOT = B * N
NW = 32
WPS = NW // B
ROWS_W = H // WPS
RCH = 32
NCH = ROWS_W // RCH
VPR = W // 16

_C0 = 0.0009251831215806305
_C1 = 0.9797525405883789
_C2 = -0.3935345709323883
_C3 = 0.10668430477380753
def _softplus_nc(xv):
    e = jnp.exp(-jnp.abs(xv))
    p = (jnp.float32(_C3) * e + jnp.float32(_C2)) * e + jnp.float32(_C1)
    return jnp.maximum(xv, jnp.float32(0.0)) + p * e


_MESH = plsc.VectorSubcoreMesh(core_axis_name="c", subcore_axis_name="s",
                               num_cores=2, num_subcores=16)


def _sc_sums_body(x_hbm, t_hbm, out_hbm, xb, tb, ob):
    c = lax.axis_index("c")
    s = lax.axis_index("s")
    wid = s * 2 + c
    samp = wid // WPS
    row0 = (wid % WPS) * ROWS_W

    def chunk_loop(ci, carry):
        acc_l, acc_t = carry
        r = row0 + ci * RCH
        pltpu.sync_copy(x_hbm.at[samp, 0, pl.ds(r, RCH), :], xb)
        pltpu.sync_copy(t_hbm.at[samp, 0, pl.ds(r, RCH), :], tb)

        def row_loop(ri, carry2):
            def vloop(i, carry3):
                al, at = carry3
                xv = xb[ri, pl.ds(i * 16, 16)]
                tv = tb[ri, pl.ds(i * 16, 16)]
                sp = _softplus_nc(xv)
                return al + (sp - xv * tv), at + tv

            return lax.fori_loop(0, VPR, vloop, carry2, unroll=8)

        return lax.fori_loop(0, RCH, row_loop, (acc_l, acc_t))

    z = jnp.zeros((16,), jnp.float32)
    acc_l, acc_t = lax.fori_loop(0, NCH, chunk_loop, (z, z))
    ob[0, :] = acc_l
    ob[1, :] = acc_t
    pltpu.sync_copy(ob, out_hbm.at[wid])


RRCH = 32
RNCH = H // RRCH


def _key_u32(xv, tv):
    b = lax.bitcast_convert_type(xv, jnp.uint32)
    neg = (b >> jnp.uint32(31)) == jnp.uint32(1)
    key = jnp.where(neg, ~b, b | jnp.uint32(0x80000000))
    key = jnp.maximum(key, jnp.uint32(1))
    return jnp.where(tv == jnp.float32(1.0), jnp.uint32(0), key)


def _lane_sum(v):
    idx = lax.iota(jnp.int32, 16)
    for sh in (1, 2, 4, 8):
        rot = v.at[(idx + sh) & 15].get(mode="promise_in_bounds")
        v = v + rot
    return v


def _sc_topk_body(x_hbm, t_hbm, out_hbm, xb, tb, ob):
    c = lax.axis_index("c")
    s = lax.axis_index("s")
    wid = s * 2 + c
    samp = jnp.minimum(wid, B - 1)
    zi = jnp.zeros((16,), jnp.int32)
    zf = jnp.zeros((16,), jnp.float32)

    def count_pass(cand_incl):
        def chunk_loop(ci, acc):
            pltpu.sync_copy(x_hbm.at[samp, 0, pl.ds(ci * RRCH, RRCH), :], xb)
            pltpu.sync_copy(t_hbm.at[samp, 0, pl.ds(ci * RRCH, RRCH), :], tb)

            def row_loop(ri, a0):
                def vloop(i, a):
                    key = _key_u32(xb[ri, pl.ds(i * 16, 16)],
                                   tb[ri, pl.ds(i * 16, 16)])
                    return a + jnp.where(key >= cand_incl, jnp.int32(1),
                                         jnp.int32(0))

                return lax.fori_loop(0, VPR, vloop, a0, unroll=4)

            return lax.fori_loop(0, RRCH, row_loop, acc)

        acc = lax.fori_loop(0, RNCH, chunk_loop, zi)
        return _lane_sum(acc)

    def nw_chunk(ci, acc):
        pltpu.sync_copy(t_hbm.at[samp, 0, pl.ds(ci * RRCH, RRCH), :], tb)

        def row_loop(ri, a0):
            def vloop(i, a):
                return a + tb[ri, pl.ds(i * 16, 16)]

            return lax.fori_loop(0, VPR, vloop, a0, unroll=4)

        return lax.fori_loop(0, RRCH, row_loop, acc)

    nwv = lax.fori_loop(0, RNCH, nw_chunk, zf)
    n_white_f = _lane_sum(nwv)
    n_white = n_white_f.astype(jnp.int32)
    n_black = jnp.full((16,), N, jnp.int32) - n_white
    k = jnp.minimum(3 * n_white, n_black)

    def bit_step(j, prefix):
        bit = jnp.full((16,), 1, jnp.uint32) << (
            jnp.uint32(31) - j.astype(jnp.uint32))
        cand = prefix | bit
        cnt = count_pass(cand)
        return jnp.where(cnt >= k, cand, prefix)

    T = lax.fori_loop(0, 32, bit_step, jnp.zeros((16,), jnp.uint32))

    def fin_chunk(ci, carry):
        pltpu.sync_copy(x_hbm.at[samp, 0, pl.ds(ci * RRCH, RRCH), :], xb)
        pltpu.sync_copy(t_hbm.at[samp, 0, pl.ds(ci * RRCH, RRCH), :], tb)

        def row_loop(ri, c0):
            def vloop(i, cc):
                aw, abs_, abc = cc
                xv = xb[ri, pl.ds(i * 16, 16)]
                tv = tb[ri, pl.ds(i * 16, 16)]
                sp = _softplus_nc(xv) + jnp.float32(_C0)
                key = _key_u32(xv, tv)
                white = tv == jnp.float32(1.0)
                gt = key > T
                aw = aw + jnp.where(white, sp - xv, jnp.float32(0.0))
                abs_ = abs_ + jnp.where(gt, sp, jnp.float32(0.0))
                abc = abc + jnp.where(gt, jnp.float32(1.0), jnp.float32(0.0))
                return aw, abs_, abc

            return lax.fori_loop(0, VPR, vloop, c0, unroll=4)

        return lax.fori_loop(0, RRCH, row_loop, carry)

    aw, abs_, abc = lax.fori_loop(0, RNCH, fin_chunk, (zf, zf, zf))
    sum_white = _lane_sum(aw)
    sum_gt = _lane_sum(abs_)
    cnt_gt = _lane_sum(abc)

    tbits = jnp.where(T >= jnp.uint32(0x80000000), T & jnp.uint32(0x7FFFFFFF),
                      ~T)
    sp_tie = _softplus_nc(lax.bitcast_convert_type(tbits, jnp.float32)) + \
        jnp.float32(_C0)
    n_tie = k.astype(jnp.float32) - cnt_gt
    sum_black = sum_gt + jnp.where(k > 0, n_tie * sp_tie, jnp.float32(0.0))

    ob[0, :] = sum_white + sum_black
    ob[1, :] = n_white_f + k.astype(jnp.float32)
    ob[2, :] = zf
    ob[3, :] = zf
    pltpu.sync_copy(ob, out_hbm.at[wid])


def _build_kernels(interpret=False):
    sums = pl.kernel(
        _sc_sums_body,
        out_type=jax.ShapeDtypeStruct((NW, 2, 16), jnp.float32),
        mesh=_MESH,
        scratch_types=[
            pltpu.VMEM((RCH, W), jnp.float32),
            pltpu.VMEM((RCH, W), jnp.float32),
            pltpu.VMEM((2, 16), jnp.float32),
        ],
        interpret=interpret,
    )
    topk = pl.kernel(
        _sc_topk_body,
        out_type=jax.ShapeDtypeStruct((NW, 4, 16), jnp.float32),
        mesh=_MESH,
        scratch_types=[
            pltpu.VMEM((RRCH, W), jnp.float32),
            pltpu.VMEM((RRCH, W), jnp.float32),
            pltpu.VMEM((4, 16), jnp.float32),
        ],
        interpret=interpret,
    )
    return sums, topk


_sc_sums, _sc_topk = _build_kernels()


def kernel(input, target):
    parts = _sc_sums(input, target)
    per_worker = jnp.sum(parts, axis=2)
    pw = per_worker.reshape(B, WPS, 2)
    loss_sum = jnp.sum(pw[:, :, 0]) + jnp.float32(NTOT * _C0)
    n_white = jnp.sum(pw[:, :, 1], axis=1)
    n_black = jnp.float32(N) - n_white
    common = loss_sum / jnp.float32(NTOT)

    def rare():
        out = _sc_topk(input, target)
        sums = out[:B, 0, 0]
        cnts = out[:B, 1, 0]
        return jnp.sum(sums) / jnp.sum(cnts)

    pred = jnp.all(3.0 * n_white >= n_black)
    return lax.cond(pred, lambda: common, rare)


# speedup vs baseline: 51.1251x; 1.1536x over previous
import functools

import jax
import jax.numpy as jnp
from jax import lax
from jax.experimental import pallas as pl
from jax.experimental.pallas import tpu as pltpu
from jax.experimental.pallas import tpu_sc as plsc

B = 8
H = 512
W = 512
N = H * W
NTOT = B * N
NW = 32
WPS = NW // B
ROWS_W = H // WPS
RCH = 32
NCH = ROWS_W // RCH
VPR = W // 16

_C0 = 0.0009251831215806305
_C1 = 0.9797525405883789
_C2 = -0.3935345709323883
_C3 = 0.10668430477380753
def _softplus_nc(xv):
    e = jnp.exp(-jnp.abs(xv))
    p = (jnp.float32(_C3) * e + jnp.float32(_C2)) * e + jnp.float32(_C1)
    return jnp.maximum(xv, jnp.float32(0.0)) + p * e


_MESH = plsc.VectorSubcoreMesh(core_axis_name="c", subcore_axis_name="s",
                               num_cores=2, num_subcores=16)


def _sc_sums_body(x_hbm, t_hbm, out_hbm, xb0, tb0, xb1, tb1, ob,
                  sx0, st0, sx1, st1):
    c = lax.axis_index("c")
    s = lax.axis_index("s")
    wid = s * 2 + c
    samp = wid // WPS
    row0 = (wid % WPS) * ROWS_W
    xbufs, tbufs = (xb0, xb1), (tb0, tb1)
    sxs, sts = (sx0, sx1), (st0, st1)

    def start(ci):
        r = row0 + ci * RCH
        b = ci & 1
        hx = pltpu.async_copy(x_hbm.at[samp, 0, pl.ds(r, RCH), :],
                              xbufs[b], sxs[b])
        ht = pltpu.async_copy(t_hbm.at[samp, 0, pl.ds(r, RCH), :],
                              tbufs[b], sts[b])
        return hx, ht

    def compute_chunk(xb, tb, carry):
        def row_loop(ri, carry2):
            def vloop(i, carry3):
                al, at = carry3
                xv = xb[ri, pl.ds(i * 16, 16)]
                tv = tb[ri, pl.ds(i * 16, 16)]
                sp = _softplus_nc(xv)
                return al + (sp - xv * tv), at + tv

            return lax.fori_loop(0, VPR, vloop, carry2, unroll=8)

        return lax.fori_loop(0, RCH, row_loop, carry)

    z = jnp.zeros((16,), jnp.float32)
    accs = (z, z)
    h = start(0)
    for ci in range(NCH):
        hx, ht = h
        if ci + 1 < NCH:
            h = start(ci + 1)
        hx.wait()
        ht.wait()
        b = ci & 1
        accs = compute_chunk(xbufs[b], tbufs[b], accs)
    acc_l, acc_t = accs
    ob[0, :] = acc_l
    ob[1, :] = acc_t
    pltpu.sync_copy(ob, out_hbm.at[wid])


RRCH = 32
RNCH = H // RRCH


def _key_u32(xv, tv):
    b = lax.bitcast_convert_type(xv, jnp.uint32)
    neg = (b >> jnp.uint32(31)) == jnp.uint32(1)
    key = jnp.where(neg, ~b, b | jnp.uint32(0x80000000))
    key = jnp.maximum(key, jnp.uint32(1))
    return jnp.where(tv == jnp.float32(1.0), jnp.uint32(0), key)


def _lane_sum(v):
    idx = lax.iota(jnp.int32, 16)
    for sh in (1, 2, 4, 8):
        rot = v.at[(idx + sh) & 15].get(mode="promise_in_bounds")
        v = v + rot
    return v


def _sc_topk_body(x_hbm, t_hbm, out_hbm, xb, tb, ob):
    c = lax.axis_index("c")
    s = lax.axis_index("s")
    wid = s * 2 + c
    samp = jnp.minimum(wid, B - 1)
    zi = jnp.zeros((16,), jnp.int32)
    zf = jnp.zeros((16,), jnp.float32)

    def count_pass(cand_incl):
        def chunk_loop(ci, acc):
            pltpu.sync_copy(x_hbm.at[samp, 0, pl.ds(ci * RRCH, RRCH), :], xb)
            pltpu.sync_copy(t_hbm.at[samp, 0, pl.ds(ci * RRCH, RRCH), :], tb)

            def row_loop(ri, a0):
                def vloop(i, a):
                    key = _key_u32(xb[ri, pl.ds(i * 16, 16)],
                                   tb[ri, pl.ds(i * 16, 16)])
                    return a + jnp.where(key >= cand_incl, jnp.int32(1),
                                         jnp.int32(0))

                return lax.fori_loop(0, VPR, vloop, a0, unroll=4)

            return lax.fori_loop(0, RRCH, row_loop, acc)

        acc = lax.fori_loop(0, RNCH, chunk_loop, zi)
        return _lane_sum(acc)

    def nw_chunk(ci, acc):
        pltpu.sync_copy(t_hbm.at[samp, 0, pl.ds(ci * RRCH, RRCH), :], tb)

        def row_loop(ri, a0):
            def vloop(i, a):
                return a + tb[ri, pl.ds(i * 16, 16)]

            return lax.fori_loop(0, VPR, vloop, a0, unroll=4)

        return lax.fori_loop(0, RRCH, row_loop, acc)

    nwv = lax.fori_loop(0, RNCH, nw_chunk, zf)
    n_white_f = _lane_sum(nwv)
    n_white = n_white_f.astype(jnp.int32)
    n_black = jnp.full((16,), N, jnp.int32) - n_white
    k = jnp.minimum(3 * n_white, n_black)

    def bit_step(j, prefix):
        bit = jnp.full((16,), 1, jnp.uint32) << (
            jnp.uint32(31) - j.astype(jnp.uint32))
        cand = prefix | bit
        cnt = count_pass(cand)
        return jnp.where(cnt >= k, cand, prefix)

    T = lax.fori_loop(0, 32, bit_step, jnp.zeros((16,), jnp.uint32))

    def fin_chunk(ci, carry):
        pltpu.sync_copy(x_hbm.at[samp, 0, pl.ds(ci * RRCH, RRCH), :], xb)
        pltpu.sync_copy(t_hbm.at[samp, 0, pl.ds(ci * RRCH, RRCH), :], tb)

        def row_loop(ri, c0):
            def vloop(i, cc):
                aw, abs_, abc = cc
                xv = xb[ri, pl.ds(i * 16, 16)]
                tv = tb[ri, pl.ds(i * 16, 16)]
                sp = _softplus_nc(xv) + jnp.float32(_C0)
                key = _key_u32(xv, tv)
                white = tv == jnp.float32(1.0)
                gt = key > T
                aw = aw + jnp.where(white, sp - xv, jnp.float32(0.0))
                abs_ = abs_ + jnp.where(gt, sp, jnp.float32(0.0))
                abc = abc + jnp.where(gt, jnp.float32(1.0), jnp.float32(0.0))
                return aw, abs_, abc

            return lax.fori_loop(0, VPR, vloop, c0, unroll=4)

        return lax.fori_loop(0, RRCH, row_loop, carry)

    aw, abs_, abc = lax.fori_loop(0, RNCH, fin_chunk, (zf, zf, zf))
    sum_white = _lane_sum(aw)
    sum_gt = _lane_sum(abs_)
    cnt_gt = _lane_sum(abc)

    tbits = jnp.where(T >= jnp.uint32(0x80000000), T & jnp.uint32(0x7FFFFFFF),
                      ~T)
    sp_tie = _softplus_nc(lax.bitcast_convert_type(tbits, jnp.float32)) + \
        jnp.float32(_C0)
    n_tie = k.astype(jnp.float32) - cnt_gt
    sum_black = sum_gt + jnp.where(k > 0, n_tie * sp_tie, jnp.float32(0.0))

    ob[0, :] = sum_white + sum_black
    ob[1, :] = n_white_f + k.astype(jnp.float32)
    ob[2, :] = zf
    ob[3, :] = zf
    pltpu.sync_copy(ob, out_hbm.at[wid])


def _build_kernels(interpret=False):
    sums = pl.kernel(
        _sc_sums_body,
        out_type=jax.ShapeDtypeStruct((NW, 2, 16), jnp.float32),
        mesh=_MESH,
        scratch_types=[
            pltpu.VMEM((RCH, W), jnp.float32),
            pltpu.VMEM((RCH, W), jnp.float32),
            pltpu.VMEM((RCH, W), jnp.float32),
            pltpu.VMEM((RCH, W), jnp.float32),
            pltpu.VMEM((2, 16), jnp.float32),
            pltpu.SemaphoreType.DMA,
            pltpu.SemaphoreType.DMA,
            pltpu.SemaphoreType.DMA,
            pltpu.SemaphoreType.DMA,
        ],
        interpret=interpret,
    )
    topk = pl.kernel(
        _sc_topk_body,
        out_type=jax.ShapeDtypeStruct((NW, 4, 16), jnp.float32),
        mesh=_MESH,
        scratch_types=[
            pltpu.VMEM((RRCH, W), jnp.float32),
            pltpu.VMEM((RRCH, W), jnp.float32),
            pltpu.VMEM((4, 16), jnp.float32),
        ],
        interpret=interpret,
    )
    return sums, topk


_sc_sums, _sc_topk = _build_kernels()


def kernel(input, target):
    parts = _sc_sums(input, target)
    per_worker = jnp.sum(parts, axis=2)
    pw = per_worker.reshape(B, WPS, 2)
    loss_sum = jnp.sum(pw[:, :, 0]) + jnp.float32(NTOT * _C0)
    n_white = jnp.sum(pw[:, :, 1], axis=1)
    n_black = jnp.float32(N) - n_white
    common = loss_sum / jnp.float32(NTOT)

    def rare():
        out = _sc_topk(input, target)
        sums = out[:B, 0, 0]
        cnts = out[:B, 1, 0]
        return jnp.sum(sums) / jnp.sum(cnts)

    pred = jnp.all(3.0 * n_white >= n_black)
    return lax.cond(pred, lambda: common, rare)
